# baseline (speedup 1.0000x reference)
import jax
import jax.numpy as jnp
from jax import lax
from jax.experimental import pallas as pl
from jax.experimental.pallas import tpu as pltpu
from jax.experimental.pallas import tpu_sc as plsc

B = 4096
L = 200
D = 300
DP = 384
LANES = 16
NC = 2
NS = 16
NW = NC * NS
BPW = B // NW
LA = 104
LB = L - LA
NCH = DP // LANES


def _zero_accs():
    return tuple(jnp.zeros((LANES,), jnp.float32) for _ in range(NCH))


def _reduce_into(buf, n, accs):
    def red(i, accs):
        r = 2 * i
        return tuple(a + (buf[r, pl.ds(o * LANES, LANES)]
                          + buf[r + 1, pl.ds(o * LANES, LANES)])
                     for o, a in enumerate(accs))
    return lax.fori_loop(0, n // 2, red, accs)


def _pool_body(x_hbm, emb_hbm, out_hbm, idx_a, idx_b, buf_a, buf_b, out_v,
               sem_a, sem_b):
    cid = lax.axis_index("c")
    sid = lax.axis_index("s")
    wid = sid * NC + cid
    base = wid * BPW

    def fire_a(idx_ref):
        pltpu.async_copy(emb_hbm.at[idx_ref.at[pl.ds(0, LA)]], buf_a, sem_a)

    def fire_b(idx_ref):
        pltpu.async_copy(emb_hbm.at[idx_ref.at[pl.ds(LA, LB)]], buf_b, sem_b)

    def wait_a():
        pltpu.make_async_copy(
            emb_hbm.at[idx_a.at[pl.ds(0, LA)]], buf_a, sem_a).wait()

    def wait_b():
        pltpu.make_async_copy(
            emb_hbm.at[idx_a.at[pl.ds(LA, LB)]], buf_b, sem_b).wait()

    def stage(b, idx_ref):
        pltpu.sync_copy(x_hbm.at[pl.ds((base + b) * L, L)], idx_ref)

    def store_row(b, accs):
        for o, a in enumerate(accs):
            out_v[b, pl.ds(o * LANES, LANES)] = a

    stage(0, idx_a)
    fire_a(idx_a)

    def per_pair(i, carry):
        r0 = 2 * i
        r1 = r0 + 1
        fire_b(idx_a)
        wait_a()
        accs = _reduce_into(buf_a, LA, _zero_accs())
        stage(r1, idx_b)
        fire_a(idx_b)
        wait_b()
        accs = _reduce_into(buf_b, LB, accs)
        store_row(r0, accs)

        fire_b(idx_b)
        wait_a()
        accs = _reduce_into(buf_a, LA, _zero_accs())

        @pl.when(i + 1 < BPW // 2)
        def _():
            stage(r1 + 1, idx_a)
            fire_a(idx_a)

        wait_b()
        accs = _reduce_into(buf_b, LB, accs)
        store_row(r1, accs)
        return carry

    lax.fori_loop(0, BPW // 2, per_pair, 0)
    pltpu.sync_copy(out_v, out_hbm.at[pl.ds(base, BPW)])


def _pool(x_flat, emb):
    mesh = plsc.VectorSubcoreMesh(core_axis_name="c", subcore_axis_name="s")
    return pl.kernel(
        _pool_body,
        out_type=jax.ShapeDtypeStruct((B, DP), jnp.float32),
        mesh=mesh,
        scratch_types=[
            pltpu.VMEM((L,), jnp.int32),
            pltpu.VMEM((L,), jnp.int32),
            pltpu.VMEM((LA, DP), jnp.float32),
            pltpu.VMEM((LB, DP), jnp.float32),
            pltpu.VMEM((BPW, DP), jnp.float32),
            pltpu.SemaphoreType.DMA,
            pltpu.SemaphoreType.DMA,
        ],
        compiler_params=pltpu.CompilerParams(use_tc_tiling_on_sc=True),
    )(x_flat, emb)


def _mlp_body(sum_ref, len_ref, w1_ref, b1_ref, w2_ref, b2_ref, w3_ref,
              b3_ref, out_ref):
    x = sum_ref[:, :D] / len_ref[...]
    dn = (((1,), (1,)), ((), ()))
    bf = jnp.bfloat16
    h1 = jax.nn.sigmoid(
        lax.dot_general(x.astype(bf), w1_ref[...].astype(bf), dn,
                        preferred_element_type=jnp.float32) + b1_ref[...])
    h2 = jax.nn.sigmoid(
        lax.dot_general(h1.astype(bf), w2_ref[...].astype(bf), dn,
                        preferred_element_type=jnp.float32) + b2_ref[...])
    w3b = w3_ref[...].astype(bf).astype(jnp.float32)
    out_ref[...] = (jnp.sum(h2.astype(bf).astype(jnp.float32) * w3b,
                            axis=1, keepdims=True) + b3_ref[0, 0])


def _mlp(pooled_sum, lengths_f, W1, b1, W2, b2, W3, b3):
    return pl.pallas_call(
        _mlp_body,
        out_shape=jax.ShapeDtypeStruct((B, 1), jnp.float32),
    )(pooled_sum, lengths_f, W1, b1.reshape(1, -1), W2, b2.reshape(1, -1),
      W3, b3.reshape(1, -1))


def kernel(x, lengths, emb, W1, b1, W2, b2, W3, b3):
    emb_p = jnp.pad(emb, ((0, 0), (0, DP - D)))
    x_flat = x.reshape(-1).astype(jnp.int32)
    pooled_sum = _pool(x_flat, emb_p)
    lengths_f = lengths.astype(jnp.float32).reshape(B, 1)
    return _mlp(pooled_sum, lengths_f, W1, b1, W2, b2, W3, b3)


# speedup vs baseline: 1.0555x; 1.0555x over previous
import jax
import jax.numpy as jnp
from jax import lax
from jax.experimental import pallas as pl
from jax.experimental.pallas import tpu as pltpu
from jax.experimental.pallas import tpu_sc as plsc

B = 4096
L = 200
D = 300
DP = 384
LANES = 16
NC = 2
NS = 16
NW = NC * NS
BPW = B // NW
CHN = (56, 48, 48, 48)
OFF = (0, 56, 104, 152)
NCH = DP // LANES


def _zero_accs():
    return tuple(jnp.zeros((LANES,), jnp.float32) for _ in range(NCH))


def _reduce_into(buf, n, accs):
    def red(i, accs):
        r = 2 * i
        return tuple(a + (buf[r, pl.ds(o * LANES, LANES)]
                          + buf[r + 1, pl.ds(o * LANES, LANES)])
                     for o, a in enumerate(accs))
    return lax.fori_loop(0, n // 2, red, accs)


def _pool_body(x_hbm, emb_hbm, out_hbm, idx_a, idx_b, bf0, bf1, bf2, bf3,
               out_v, s0, s1, s2, s3):
    cid = lax.axis_index("c")
    sid = lax.axis_index("s")
    wid = sid * NC + cid
    base = wid * BPW
    bufs = (bf0, bf1, bf2, bf3)
    sems = (s0, s1, s2, s3)

    def fire(k, idx_ref):
        pltpu.async_copy(
            emb_hbm.at[idx_ref.at[pl.ds(OFF[k], CHN[k])]], bufs[k], sems[k])

    def wait(k):
        pltpu.make_async_copy(
            emb_hbm.at[idx_a.at[pl.ds(OFF[k], CHN[k])]], bufs[k],
            sems[k]).wait()

    def stage(b, idx_ref):
        pltpu.sync_copy(x_hbm.at[pl.ds((base + b) * L, L)], idx_ref)

    def store_row(b, accs):
        for o, a in enumerate(accs):
            out_v[b, pl.ds(o * LANES, LANES)] = a

    def do_row(r, idx_cur, idx_nxt, stage_next):
        @pl.when(stage_next)
        def _():
            stage(r + 1, idx_nxt)
        accs = _zero_accs()
        for k in range(4):
            wait(k)
            accs = _reduce_into(bufs[k], CHN[k], accs)

            @pl.when(stage_next)
            def _():
                fire(k, idx_nxt)
        store_row(r, accs)

    stage(0, idx_a)
    for k in range(4):
        fire(k, idx_a)

    def per_pair(i, carry):
        r0 = 2 * i
        do_row(r0, idx_a, idx_b, r0 + 1 < BPW)
        do_row(r0 + 1, idx_b, idx_a, r0 + 2 < BPW)
        return carry

    lax.fori_loop(0, BPW // 2, per_pair, 0)
    pltpu.sync_copy(out_v, out_hbm.at[pl.ds(base, BPW)])


def _pool(x_flat, emb):
    mesh = plsc.VectorSubcoreMesh(core_axis_name="c", subcore_axis_name="s")
    return pl.kernel(
        _pool_body,
        out_type=jax.ShapeDtypeStruct((B, DP), jnp.float32),
        mesh=mesh,
        scratch_types=[
            pltpu.VMEM((L,), jnp.int32),
            pltpu.VMEM((L,), jnp.int32),
            pltpu.VMEM((CHN[0], DP), jnp.float32),
            pltpu.VMEM((CHN[1], DP), jnp.float32),
            pltpu.VMEM((CHN[2], DP), jnp.float32),
            pltpu.VMEM((CHN[3], DP), jnp.float32),
            pltpu.VMEM((BPW, DP), jnp.float32),
            pltpu.SemaphoreType.DMA,
            pltpu.SemaphoreType.DMA,
            pltpu.SemaphoreType.DMA,
            pltpu.SemaphoreType.DMA,
        ],
        compiler_params=pltpu.CompilerParams(use_tc_tiling_on_sc=True),
    )(x_flat, emb)


def _mlp_body(sum_ref, len_ref, w1_ref, b1_ref, w2_ref, b2_ref, w3_ref,
              b3_ref, out_ref):
    x = sum_ref[:, :D] / len_ref[...]
    dn = (((1,), (1,)), ((), ()))
    bf = jnp.bfloat16
    h1 = jax.nn.sigmoid(
        lax.dot_general(x.astype(bf), w1_ref[...].astype(bf), dn,
                        preferred_element_type=jnp.float32) + b1_ref[...])
    h2 = jax.nn.sigmoid(
        lax.dot_general(h1.astype(bf), w2_ref[...].astype(bf), dn,
                        preferred_element_type=jnp.float32) + b2_ref[...])
    w3b = w3_ref[...].astype(bf).astype(jnp.float32)
    out_ref[...] = (jnp.sum(h2.astype(bf).astype(jnp.float32) * w3b,
                            axis=1, keepdims=True) + b3_ref[0, 0])


def _mlp(pooled_sum, lengths_f, W1, b1, W2, b2, W3, b3):
    return pl.pallas_call(
        _mlp_body,
        out_shape=jax.ShapeDtypeStruct((B, 1), jnp.float32),
    )(pooled_sum, lengths_f, W1, b1.reshape(1, -1), W2, b2.reshape(1, -1),
      W3, b3.reshape(1, -1))


def kernel(x, lengths, emb, W1, b1, W2, b2, W3, b3):
    emb_p = jnp.pad(emb, ((0, 0), (0, DP - D)))
    x_flat = x.reshape(-1).astype(jnp.int32)
    pooled_sum = _pool(x_flat, emb_p)
    lengths_f = lengths.astype(jnp.float32).reshape(B, 1)
    return _mlp(pooled_sum, lengths_f, W1, b1, W2, b2, W3, b3)
